# baseline (speedup 1.0000x reference)
import functools

import jax
import jax.numpy as jnp
from jax import lax
from jax.experimental import pallas as pl
from jax.experimental.pallas import tpu as pltpu
from jax.experimental.pallas import tpu_sc as plsc

_LANES = 16
_ROW = 128
_CHUNK = 128
_UNROLL = 4


@functools.lru_cache(maxsize=None)
def _make_sc_permute(n_rows: int):
    info = plsc.get_sparse_core_info()
    nc, ns = info.num_cores, info.num_subcores
    nw = nc * ns
    rows_per_w = n_rows // nw
    assert rows_per_w * nw == n_rows, (n_rows, nw)
    nch = rows_per_w // _CHUNK
    assert nch * _CHUNK == rows_per_w and nch >= 4 and nch % 2 == 0, nch
    ngrp = _ROW // _LANES

    mesh = plsc.VectorSubcoreMesh(core_axis_name="c", subcore_axis_name="s")

    @functools.partial(
        pl.kernel,
        mesh=mesh,
        out_type=jax.ShapeDtypeStruct((n_rows, _ROW), jnp.float32),
        compiler_params=pltpu.CompilerParams(needs_layout_passes=False),
        scratch_types=[
            pltpu.VMEM((_ROW,), jnp.int32),
            pltpu.VMEM((_CHUNK, _ROW), jnp.float32),
            pltpu.VMEM((_CHUNK, _ROW), jnp.float32),
            pltpu.VMEM((_CHUNK, _ROW), jnp.float32),
            pltpu.VMEM((_CHUNK, _ROW), jnp.float32),
            pltpu.SemaphoreType.DMA,
            pltpu.SemaphoreType.DMA,
            pltpu.SemaphoreType.DMA,
            pltpu.SemaphoreType.DMA,
        ],
    )
    def permute_kernel(x_hbm, perm_hbm, out_hbm, perm_v, in0, in1, out0, out1,
                       isem0, isem1, osem0, osem1):
        wid = lax.axis_index("s") * nc + lax.axis_index("c")
        base = wid * rows_per_w

        pltpu.sync_copy(perm_hbm, perm_v)
        pvecs = [perm_v[pl.ds(g * _LANES, _LANES)] for g in range(ngrp)]

        inb = (in0, in1)
        outb = (out0, out1)
        isem = (isem0, isem1)
        osem = (osem0, osem1)

        def start_in(c, b):
            pltpu.async_copy(
                x_hbm.at[pl.ds(base + c * _CHUNK, _CHUNK), :], inb[b], isem[b])

        def wait_in(b):
            pltpu.make_async_copy(
                x_hbm.at[pl.ds(base, _CHUNK), :], inb[b], isem[b]).wait()

        def start_out(c, b):
            pltpu.async_copy(
                outb[b], out_hbm.at[pl.ds(base + c * _CHUNK, _CHUNK), :], osem[b])

        def wait_out(b):
            pltpu.make_async_copy(
                outb[b], out_hbm.at[pl.ds(base, _CHUNK), :], osem[b]).wait()

        def compute(b):
            src = inb[b]
            dst = outb[b]

            def row_body(i, carry):
                for u in range(_UNROLL):
                    r = i * _UNROLL + u
                    ridx = jnp.full((_LANES,), r, jnp.int32)
                    for g in range(ngrp):
                        dst[r, pl.ds(g * _LANES, _LANES)] = plsc.load_gather(
                            src, [ridx, pvecs[g]])
                return carry
            lax.fori_loop(0, _CHUNK // _UNROLL, row_body, 0)

        start_in(0, 0)
        start_in(1, 1)
        for b in (0, 1):
            wait_in(b)
            compute(b)
            start_out(b, b)
            start_in(b + 2, b)

        def pair_body(p, carry):
            c0 = p * 2
            for b in (0, 1):
                wait_in(b)
                wait_out(b)
                compute(b)
                start_out(c0 + b, b)
                start_in(c0 + b + 2, b)
            return carry
        lax.fori_loop(1, nch // 2 - 1, pair_body, 0)

        for b in (0, 1):
            wait_in(b)
            wait_out(b)
            compute(b)
            start_out(nch - 2 + b, b)
        wait_out(0)
        wait_out(1)

    return permute_kernel


def kernel(input, permutation):
    shape = input.shape
    x2 = input.reshape(-1, shape[-1])
    out2 = _make_sc_permute(x2.shape[0])(x2, permutation)
    return out2.reshape(shape)


# speedup vs baseline: 1.6571x; 1.6571x over previous
import functools

import jax
import jax.numpy as jnp
from jax import lax
from jax.experimental import pallas as pl
from jax.experimental.pallas import tpu as pltpu
from jax.experimental.pallas import tpu_sc as plsc

_LANES = 16
_ROW = 128
_CB = 4
_SPAD = 56


@functools.lru_cache(maxsize=None)
def _make_sc_permute(n_batch: int, seq: int):
    info = plsc.get_sparse_core_info()
    nc, ns = info.num_cores, info.num_subcores
    nw = nc * ns
    bat_per_w = n_batch // nw
    assert bat_per_w * nw == n_batch, (n_batch, nw)
    nch = bat_per_w // _CB
    assert nch * _CB == bat_per_w and nch >= 4 and nch % 2 == 0, nch
    assert seq <= _SPAD
    ngrp = _ROW // _LANES

    mesh = plsc.VectorSubcoreMesh(core_axis_name="c", subcore_axis_name="s")

    @functools.partial(
        pl.kernel,
        mesh=mesh,
        out_type=jax.ShapeDtypeStruct((n_batch, seq, _ROW), jnp.float32),
        compiler_params=pltpu.CompilerParams(
            needs_layout_passes=False, use_tc_tiling_on_sc=True),
        scratch_types=[
            pltpu.VMEM((_ROW,), jnp.int32),
            pltpu.VMEM((_CB, _SPAD, _ROW), jnp.float32),
            pltpu.VMEM((_CB, _SPAD, _ROW), jnp.float32),
            pltpu.VMEM((_CB, _SPAD, _ROW), jnp.float32),
            pltpu.VMEM((_CB, _SPAD, _ROW), jnp.float32),
            pltpu.SemaphoreType.DMA,
            pltpu.SemaphoreType.DMA,
            pltpu.SemaphoreType.DMA,
            pltpu.SemaphoreType.DMA,
        ],
    )
    def permute_kernel(x_hbm, perm_hbm, out_hbm, perm_v, in0, in1, out0, out1,
                       isem0, isem1, osem0, osem1):
        wid = lax.axis_index("s") * nc + lax.axis_index("c")
        base = wid * bat_per_w

        pltpu.sync_copy(perm_hbm, perm_v)
        pvecs = [perm_v[pl.ds(g * _LANES, _LANES)] for g in range(ngrp)]

        inb = (in0, in1)
        outb = (out0, out1)
        isem = (isem0, isem1)
        osem = (osem0, osem1)

        def start_in(c, b):
            pltpu.async_copy(
                x_hbm.at[pl.ds(base + c * _CB, _CB), :, :],
                inb[b].at[:, pl.ds(0, seq), :], isem[b])

        def wait_in(b):
            pltpu.make_async_copy(
                x_hbm.at[pl.ds(base, _CB), :, :],
                inb[b].at[:, pl.ds(0, seq), :], isem[b]).wait()

        def start_out(c, b):
            pltpu.async_copy(
                outb[b].at[:, pl.ds(0, seq), :],
                out_hbm.at[pl.ds(base + c * _CB, _CB), :, :], osem[b])

        def wait_out(b):
            pltpu.make_async_copy(
                outb[b].at[:, pl.ds(0, seq), :],
                out_hbm.at[pl.ds(base, _CB), :, :], osem[b]).wait()

        def compute(b):
            src = inb[b]
            dst = outb[b]

            def row_body(s, carry):
                sidx = jnp.full((_LANES,), s, jnp.int32)
                for bb in range(_CB):
                    bidx = jnp.full((_LANES,), bb, jnp.int32)
                    for g in range(ngrp):
                        dst[bb, s, pl.ds(g * _LANES, _LANES)] = (
                            plsc.load_gather(src, [bidx, sidx, pvecs[g]]))
                return carry
            lax.fori_loop(0, seq, row_body, 0)

        start_in(0, 0)
        start_in(1, 1)
        for b in (0, 1):
            wait_in(b)
            compute(b)
            start_out(b, b)
            start_in(b + 2, b)

        def pair_body(p, carry):
            c0 = p * 2
            for b in (0, 1):
                wait_in(b)
                wait_out(b)
                compute(b)
                start_out(c0 + b, b)
                start_in(c0 + b + 2, b)
            return carry
        lax.fori_loop(1, nch // 2 - 1, pair_body, 0)

        for b in (0, 1):
            wait_in(b)
            wait_out(b)
            compute(b)
            start_out(nch - 2 + b, b)
        wait_out(0)
        wait_out(1)

    return permute_kernel


def kernel(input, permutation):
    n_batch, seq, row = input.shape
    assert row == _ROW
    return _make_sc_permute(n_batch, seq)(input, permutation)


# speedup vs baseline: 2.6068x; 1.5731x over previous
import functools

import jax
import jax.numpy as jnp
from jax import lax
from jax.experimental import pallas as pl
from jax.experimental.pallas import tpu as pltpu
from jax.experimental.pallas import tpu_sc as plsc

_LANES = 16
_ROW = 128
_CB = 2
_NBUF = 4
_SPAD = 56


@functools.lru_cache(maxsize=None)
def _make_sc_permute(n_batch: int, seq: int):
    info = plsc.get_sparse_core_info()
    nc, ns = info.num_cores, info.num_subcores
    nw = nc * ns
    bat_per_w = n_batch // nw
    assert bat_per_w * nw == n_batch, (n_batch, nw)
    nch = bat_per_w // _CB
    assert nch * _CB == bat_per_w and nch % _NBUF == 0 and nch // _NBUF >= 3, nch
    assert seq <= _SPAD
    ngrp = _ROW // _LANES

    mesh = plsc.VectorSubcoreMesh(core_axis_name="c", subcore_axis_name="s")

    @functools.partial(
        pl.kernel,
        mesh=mesh,
        out_type=jax.ShapeDtypeStruct((n_batch, seq, _ROW), jnp.float32),
        compiler_params=pltpu.CompilerParams(
            needs_layout_passes=False, use_tc_tiling_on_sc=True),
        scratch_types=(
            [pltpu.VMEM((_ROW,), jnp.int32)]
            + [pltpu.VMEM((_CB, _SPAD, _ROW), jnp.float32)] * (2 * _NBUF)
            + [pltpu.SemaphoreType.DMA] * (2 * _NBUF)
        ),
    )
    def permute_kernel(x_hbm, perm_hbm, out_hbm, perm_v, *bufs_and_sems):
        inb = bufs_and_sems[:_NBUF]
        outb = bufs_and_sems[_NBUF:2 * _NBUF]
        isem = bufs_and_sems[2 * _NBUF:3 * _NBUF]
        osem = bufs_and_sems[3 * _NBUF:4 * _NBUF]
        wid = lax.axis_index("s") * nc + lax.axis_index("c")
        base = wid * bat_per_w

        pltpu.sync_copy(perm_hbm, perm_v)
        pvecs = [perm_v[pl.ds(g * _LANES, _LANES)] for g in range(ngrp)]

        def start_in(c, b):
            pltpu.async_copy(
                x_hbm.at[pl.ds(base + c * _CB, _CB), :, :],
                inb[b].at[:, pl.ds(0, seq), :], isem[b])

        def wait_in(b):
            pltpu.make_async_copy(
                x_hbm.at[pl.ds(base, _CB), :, :],
                inb[b].at[:, pl.ds(0, seq), :], isem[b]).wait()

        def start_out(c, b):
            pltpu.async_copy(
                outb[b].at[:, pl.ds(0, seq), :],
                out_hbm.at[pl.ds(base + c * _CB, _CB), :, :], osem[b])

        def wait_out(b):
            pltpu.make_async_copy(
                outb[b].at[:, pl.ds(0, seq), :],
                out_hbm.at[pl.ds(base, _CB), :, :], osem[b]).wait()

        def compute(b):
            src = inb[b]
            dst = outb[b]

            @plsc.parallel_loop(0, seq, 1, unroll=2)
            def row_body(s):
                sidx = jnp.full((_LANES,), s, jnp.int32)
                for bb in range(_CB):
                    bidx = jnp.full((_LANES,), bb, jnp.int32)
                    for g in range(ngrp):
                        dst[bb, s, pl.ds(g * _LANES, _LANES)] = (
                            plsc.load_gather(src, [bidx, sidx, pvecs[g]]))

        for b in range(_NBUF):
            start_in(b, b)
        for b in range(_NBUF):
            wait_in(b)
            compute(b)
            start_out(b, b)
            start_in(b + _NBUF, b)

        def round_body(p, carry):
            c0 = p * _NBUF
            for b in range(_NBUF):
                wait_in(b)
                wait_out(b)
                compute(b)
                start_out(c0 + b, b)
                start_in(c0 + b + _NBUF, b)
            return carry
        lax.fori_loop(1, nch // _NBUF - 1, round_body, 0)

        for b in range(_NBUF):
            wait_in(b)
            wait_out(b)
            compute(b)
            start_out(nch - _NBUF + b, b)
        for b in range(_NBUF):
            wait_out(b)

    return permute_kernel


def kernel(input, permutation):
    n_batch, seq, row = input.shape
    assert row == _ROW
    return _make_sc_permute(n_batch, seq)(input, permutation)


# speedup vs baseline: 6.3322x; 2.4291x over previous
import functools

import jax
import jax.numpy as jnp
from jax import lax
from jax.experimental import pallas as pl
from jax.experimental.pallas import tpu as pltpu
from jax.experimental.pallas import tpu_sc as plsc

_LANES = 16
_ROW = 128
_CHUNK = 128
_NBUF = 2


@functools.lru_cache(maxsize=None)
def _make_sc_permute(n_rows: int):
    info = plsc.get_sparse_core_info()
    nc, ns = info.num_cores, info.num_subcores
    nw = nc * ns
    rows_per_w = n_rows // nw
    assert rows_per_w * nw == n_rows, (n_rows, nw)
    nch = rows_per_w // _CHUNK
    assert nch * _CHUNK == rows_per_w and nch % _NBUF == 0, nch
    assert nch // _NBUF >= 3, nch
    ngrp = _ROW // _LANES

    mesh = plsc.VectorSubcoreMesh(core_axis_name="c", subcore_axis_name="s")

    @functools.partial(
        pl.kernel,
        mesh=mesh,
        out_type=jax.ShapeDtypeStruct((n_rows, _ROW), jnp.float32),
        compiler_params=pltpu.CompilerParams(needs_layout_passes=False),
        scratch_types=(
            [pltpu.VMEM((_ROW,), jnp.int32)]
            + [pltpu.VMEM((_CHUNK, _ROW), jnp.float32)] * (2 * _NBUF)
            + [pltpu.SemaphoreType.DMA] * (2 * _NBUF)
        ),
    )
    def permute_kernel(x_hbm, perm_hbm, out_hbm, perm_v, *bufs_and_sems):
        inb = bufs_and_sems[:_NBUF]
        outb = bufs_and_sems[_NBUF:2 * _NBUF]
        isem = bufs_and_sems[2 * _NBUF:3 * _NBUF]
        osem = bufs_and_sems[3 * _NBUF:4 * _NBUF]

        wid = lax.axis_index("s") * nc + lax.axis_index("c")
        base = wid * rows_per_w

        pltpu.sync_copy(perm_hbm, perm_v)
        pvecs = [perm_v[pl.ds(g * _LANES, _LANES)] for g in range(ngrp)]

        def start_in(c, b):
            pltpu.async_copy(
                x_hbm.at[pl.ds(base + c * _CHUNK, _CHUNK), :], inb[b], isem[b])

        def wait_in(b):
            pltpu.make_async_copy(
                x_hbm.at[pl.ds(base, _CHUNK), :], inb[b], isem[b]).wait()

        def start_out(c, b):
            pltpu.async_copy(
                outb[b], out_hbm.at[pl.ds(base + c * _CHUNK, _CHUNK), :], osem[b])

        def wait_out(b):
            pltpu.make_async_copy(
                outb[b], out_hbm.at[pl.ds(base, _CHUNK), :], osem[b]).wait()

        def compute(b):
            src = inb[b]
            dst = outb[b]

            @plsc.parallel_loop(0, _CHUNK, 1, unroll=2)
            def row_body(r):
                ridx = jnp.full((_LANES,), r, jnp.int32)
                for g in range(ngrp):
                    dst[r, pl.ds(g * _LANES, _LANES)] = (
                        plsc.load_gather(src, [ridx, pvecs[g]]))

        for b in range(_NBUF):
            start_in(b, b)
        for b in range(_NBUF):
            wait_in(b)
            compute(b)
            start_out(b, b)
            start_in(b + _NBUF, b)

        def round_body(p, carry):
            c0 = p * _NBUF
            for b in range(_NBUF):
                wait_in(b)
                wait_out(b)
                compute(b)
                start_out(c0 + b, b)
                start_in(c0 + b + _NBUF, b)
            return carry
        lax.fori_loop(1, nch // _NBUF - 1, round_body, 0)

        for b in range(_NBUF):
            wait_in(b)
            wait_out(b)
            compute(b)
            start_out(nch - _NBUF + b, b)
        for b in range(_NBUF):
            wait_out(b)

    return permute_kernel


def kernel(input, permutation):
    n_batch, seq, row = input.shape
    assert row == _ROW
    xt = jnp.transpose(input, (1, 0, 2)).reshape(seq * n_batch, row)
    out2 = _make_sc_permute(seq * n_batch)(xt, permutation)
    return jnp.transpose(out2.reshape(seq, n_batch, row), (1, 0, 2))
